# Initial kernel scaffold; baseline (speedup 1.0000x reference)
import functools

import jax
import jax.numpy as jnp
from jax import lax
from jax.experimental import pallas as pl
from jax.experimental.pallas import tpu as pltpu
from jax.experimental.pallas import tpu_sc as plsc

N = 10000
E = 320000
D_IN = 128
H = 256
G = 64
BN_EPS = 1e-5

NC = 2
NS = 16
EPT = E // NS
CH = 125
NCH = EPT // CH
RPT = N // NS


def _make_agg(Hc):
    mesh = plsc.VectorSubcoreMesh(core_axis_name="c", subcore_axis_name="s")

    @functools.partial(
        pl.kernel,
        mesh=mesh,
        out_type=jax.ShapeDtypeStruct((NC, N, Hc), jnp.float32),
        scratch_types=[
            pltpu.VMEM((NCH, CH), jnp.int32),
            pltpu.VMEM((NCH, CH), jnp.int32),
            pltpu.VMEM((CH, Hc), jnp.float32),
            pltpu.VMEM((CH, Hc), jnp.float32),
            pltpu.VMEM_SHARED((N, Hc), jnp.float32),
            pltpu.SemaphoreType.DMA,
            pltpu.SemaphoreType.DMA,
        ],
    )
    def agg(h_hbm, src_hbm, dst_hbm, out_hbm, src_v, dst_v, rows0, rows1,
            acc_sh, sem0, sem1):
        c = lax.axis_index("c")
        s = lax.axis_index("s")
        table = h_hbm.at[c]

        pltpu.sync_copy(src_hbm.at[s], src_v)
        pltpu.sync_copy(dst_hbm.at[s], dst_v)
        pltpu.sync_copy(table.at[pl.ds(s * RPT, RPT)],
                        acc_sh.at[pl.ds(s * RPT, RPT)])
        plsc.subcore_barrier()

        rows = (rows0, rows1)
        sems = (sem0, sem1)
        pltpu.async_copy(table.at[src_v.at[0]], rows0, sem0)
        pltpu.async_copy(table.at[src_v.at[1]], rows1, sem1)

        def body(jj, carry):
            for k in range(2):
                j = jj * 2 + k
                pltpu.make_async_copy(table.at[src_v.at[j]], rows[k],
                                      sems[k]).wait()
                pltpu.sync_copy(rows[k], acc_sh.at[dst_v.at[j]], add=True)

                @pl.when(j + 2 < NCH)
                def _():
                    pltpu.async_copy(table.at[src_v.at[j + 2]], rows[k],
                                     sems[k])
            return carry

        lax.fori_loop(0, NCH // 2, body, 0)
        plsc.subcore_barrier()
        pltpu.sync_copy(acc_sh.at[pl.ds(s * RPT, RPT)],
                        out_hbm.at[c, pl.ds(s * RPT, RPT)])

    return agg


_agg_in = _make_agg(D_IN // 2)
_agg_hid = _make_agg(H // 2)


def _mlp_call(xs, W, bvec, gvec, bevec, Hc):
    BN_ROWS = 1000
    grid = N // BN_ROWS
    inv = 1.0 / (1.0 + BN_EPS) ** 0.5
    Ho = H // 2

    def body(x_ref, w_ref, b_ref, g_ref, be_ref, o_ref):
        x0 = x_ref[0]
        x1 = x_ref[1]
        h = jnp.dot(x0, w_ref[:Hc, :], preferred_element_type=jnp.float32)
        h = h + jnp.dot(x1, w_ref[Hc:, :], preferred_element_type=jnp.float32)
        h = h + b_ref[...]
        h = g_ref[...] * (h * inv) + be_ref[...]
        h = jnp.maximum(h, 0.0)
        o_ref[0] = h[:, :Ho]
        o_ref[1] = h[:, Ho:]

    return pl.pallas_call(
        body,
        grid=(grid,),
        in_specs=[
            pl.BlockSpec((NC, BN_ROWS, Hc), lambda i: (0, i, 0)),
            pl.BlockSpec((2 * Hc, H), lambda i: (0, 0)),
            pl.BlockSpec((1, H), lambda i: (0, 0)),
            pl.BlockSpec((1, H), lambda i: (0, 0)),
            pl.BlockSpec((1, H), lambda i: (0, 0)),
        ],
        out_specs=pl.BlockSpec((NC, BN_ROWS, Ho), lambda i: (0, i, 0)),
        out_shape=jax.ShapeDtypeStruct((NC, N, Ho), jnp.float32),
    )(xs, W, bvec.reshape(1, H), gvec.reshape(1, H), bevec.reshape(1, H))


def _pool_call(xs, W_lin, b_lin, batch2d):
    BN_ROWS = 1000
    grid = N // BN_ROWS
    Hc = H // 2

    def body(x_ref, wl_ref, bl_ref, batch_ref, o_ref, acc, cnt):
        i = pl.program_id(0)

        @pl.when(i == 0)
        def _():
            acc[...] = jnp.zeros_like(acc)
            cnt[...] = jnp.zeros_like(cnt)

        z = jnp.dot(x_ref[0], wl_ref[:Hc, :],
                    preferred_element_type=jnp.float32)
        z = z + jnp.dot(x_ref[1], wl_ref[Hc:, :],
                        preferred_element_type=jnp.float32)
        ids = batch_ref[...]
        gids = lax.broadcasted_iota(jnp.int32, (BN_ROWS, G), 1)
        mask = (ids == gids).astype(jnp.float32)
        acc[...] += jnp.sum(mask * z, axis=0, keepdims=True)
        cnt[...] += jnp.sum(mask, axis=0, keepdims=True)

        @pl.when(i == grid - 1)
        def _():
            mean = acc[...] / jnp.maximum(cnt[...], 1.0)
            o_ref[...] = mean.reshape(G, 1) + bl_ref[0, 0]

    return pl.pallas_call(
        body,
        grid=(grid,),
        in_specs=[
            pl.BlockSpec((NC, BN_ROWS, Hc), lambda i: (0, i, 0)),
            pl.BlockSpec((H, 1), lambda i: (0, 0)),
            pl.BlockSpec((1, 1), lambda i: (0, 0)),
            pl.BlockSpec((BN_ROWS, 1), lambda i: (i, 0)),
        ],
        out_specs=pl.BlockSpec((G, 1), lambda i: (0, 0)),
        out_shape=jax.ShapeDtypeStruct((G, 1), jnp.float32),
        scratch_shapes=[
            pltpu.VMEM((1, G), jnp.float32),
            pltpu.VMEM((1, G), jnp.float32),
        ],
    )(xs, W_lin, b_lin.reshape(1, 1), batch2d)


def kernel(x, edge_index, batch, W0, b0, g0, be0, W1, b1, g1, be1,
           W2, b2, g2, be2, W3, b3, g3, be3, W_lin, b_lin):
    h = x.reshape(N, NC, D_IN // NC).transpose(1, 0, 2)
    src = edge_index[0].reshape(NS, NCH, CH)
    dst = edge_index[1].reshape(NS, NCH, CH)
    batch2d = batch.reshape(N, 1)

    layers = [(W0, b0, g0, be0), (W1, b1, g1, be1),
              (W2, b2, g2, be2), (W3, b3, g3, be3)]
    for li, (W, b, g, be) in enumerate(layers):
        aggf = _agg_in if li == 0 else _agg_hid
        Hc = (D_IN // 2) if li == 0 else (H // 2)
        a = aggf(h, src, dst)
        h = _mlp_call(a, W, b, g, be, Hc)

    return _pool_call(h, W_lin, b_lin, batch2d)


# speedup vs baseline: 8.6281x; 8.6281x over previous
import functools

import jax
import jax.numpy as jnp
from jax import lax
from jax.experimental import pallas as pl
from jax.experimental.pallas import tpu as pltpu
from jax.experimental.pallas import tpu_sc as plsc

N = 10000
NP = 10240
E = 320000
D_IN = 128
H = 256
G = 64
BN_EPS = 1e-5

NC = 2
NS = 16
EPT = E // NS
CH = 125
NCH = EPT // CH
NCH0 = E // (NC * NS) // CH
RPT = NP // NS


GC = 16


def _edge_loop(table, src_hbm_t, dst_hbm_t, src_v, dst_v, rows, sems,
               acc_sh, nch):
    def group(g, carry):
        pltpu.sync_copy(src_hbm_t.at[pl.ds(g * GC, GC)], src_v)
        pltpu.sync_copy(dst_hbm_t.at[pl.ds(g * GC, GC)], dst_v)
        pltpu.async_copy(table.at[src_v.at[0]], rows[0], sems[0])
        pltpu.async_copy(table.at[src_v.at[1]], rows[1], sems[1])
        for k in range(GC):
            b = k % 2
            pltpu.make_async_copy(table.at[src_v.at[k]], rows[b],
                                  sems[b]).wait()
            pltpu.sync_copy(rows[b], acc_sh.at[dst_v.at[k]], add=True)
            if k + 2 < GC:
                pltpu.async_copy(table.at[src_v.at[k + 2]], rows[b], sems[b])
        return carry

    lax.fori_loop(0, nch // GC, group, 0)


@functools.lru_cache(maxsize=None)
def _make_agg0():
    mesh = plsc.VectorSubcoreMesh(core_axis_name="c", subcore_axis_name="s")

    @functools.partial(
        pl.kernel,
        mesh=mesh,
        out_type=jax.ShapeDtypeStruct((NC, NP, D_IN), jnp.float32),
        scratch_types=[
            pltpu.VMEM((GC, CH), jnp.int32),
            pltpu.VMEM((GC, CH), jnp.int32),
            pltpu.VMEM((CH, D_IN), jnp.float32),
            pltpu.VMEM((CH, D_IN), jnp.float32),
            pltpu.VMEM_SHARED((NP, D_IN), jnp.float32),
            pltpu.SemaphoreType.DMA,
            pltpu.SemaphoreType.DMA,
        ],
    )
    def agg0(h_hbm, z_hbm, src_hbm, dst_hbm, out_hbm, src_v, dst_v,
             rows0, rows1, acc_sh, sem0, sem1):
        c = lax.axis_index("c")
        s = lax.axis_index("s")

        slab = pl.ds(s * RPT, RPT)

        @pl.when(c == 0)
        def _():
            pltpu.sync_copy(h_hbm.at[slab], acc_sh.at[slab])

        @pl.when(c == 1)
        def _():
            pltpu.sync_copy(z_hbm.at[slab], acc_sh.at[slab])

        plsc.subcore_barrier()
        _edge_loop(h_hbm, src_hbm.at[c, s], dst_hbm.at[c, s], src_v, dst_v,
                   (rows0, rows1), (sem0, sem1), acc_sh, NCH0)
        plsc.subcore_barrier()
        pltpu.sync_copy(acc_sh.at[slab], out_hbm.at[c, slab])

    return agg0


@functools.lru_cache(maxsize=None)
def _make_agg():
    Hc = H // 2
    mesh = plsc.VectorSubcoreMesh(core_axis_name="c", subcore_axis_name="s")

    @functools.partial(
        pl.kernel,
        mesh=mesh,
        out_type=jax.ShapeDtypeStruct((NC, NP, Hc), jnp.float32),
        scratch_types=[
            pltpu.VMEM((GC, CH), jnp.int32),
            pltpu.VMEM((GC, CH), jnp.int32),
            pltpu.VMEM((CH, Hc), jnp.float32),
            pltpu.VMEM((CH, Hc), jnp.float32),
            pltpu.VMEM_SHARED((NP, Hc), jnp.float32),
            pltpu.SemaphoreType.DMA,
            pltpu.SemaphoreType.DMA,
        ],
    )
    def agg(h_hbm, src_hbm, dst_hbm, out_hbm, src_v, dst_v, rows0, rows1,
            acc_sh, sem0, sem1):
        c = lax.axis_index("c")
        s = lax.axis_index("s")
        table = h_hbm.at[c]

        slab = pl.ds(s * RPT, RPT)
        pltpu.sync_copy(table.at[slab], acc_sh.at[slab])
        plsc.subcore_barrier()
        _edge_loop(table, src_hbm.at[s], dst_hbm.at[s], src_v, dst_v,
                   (rows0, rows1), (sem0, sem1), acc_sh, NCH)
        plsc.subcore_barrier()
        pltpu.sync_copy(acc_sh.at[slab], out_hbm.at[c, slab])

    return agg


def _mlp_call(xs, W, bvec, gvec, bevec, Hc, sum_planes=False):
    BN_ROWS = 1024
    grid = NP // BN_ROWS
    inv = 1.0 / (1.0 + BN_EPS) ** 0.5
    Ho = H // 2

    def body(x_ref, w_ref, b_ref, g_ref, be_ref, o_ref):
        x0 = x_ref[0]
        x1 = x_ref[1]
        if sum_planes:
            h = jnp.dot(x0 + x1, w_ref[...],
                        preferred_element_type=jnp.float32)
        else:
            h = jnp.dot(x0, w_ref[:Hc, :],
                        preferred_element_type=jnp.float32)
            h = h + jnp.dot(x1, w_ref[Hc:, :],
                            preferred_element_type=jnp.float32)
        h = h + b_ref[...]
        h = g_ref[...] * (h * inv) + be_ref[...]
        h = jnp.maximum(h, 0.0)
        o_ref[0] = h[:, :Ho]
        o_ref[1] = h[:, Ho:]

    return pl.pallas_call(
        body,
        grid=(grid,),
        in_specs=[
            pl.BlockSpec((NC, BN_ROWS, Hc), lambda i: (0, i, 0)),
            pl.BlockSpec((Hc if sum_planes else 2 * Hc, H),
                         lambda i: (0, 0)),
            pl.BlockSpec((1, H), lambda i: (0, 0)),
            pl.BlockSpec((1, H), lambda i: (0, 0)),
            pl.BlockSpec((1, H), lambda i: (0, 0)),
        ],
        out_specs=pl.BlockSpec((NC, BN_ROWS, Ho), lambda i: (0, i, 0)),
        out_shape=jax.ShapeDtypeStruct((NC, NP, Ho), jnp.float32),
    )(xs, W, bvec.reshape(1, H), gvec.reshape(1, H), bevec.reshape(1, H))


def _pool_call(xs, W_lin, b_lin, batch2d):
    BN_ROWS = 1024
    grid = NP // BN_ROWS
    Hc = H // 2

    def body(x_ref, wl_ref, bl_ref, batch_ref, o_ref, acc, cnt):
        i = pl.program_id(0)

        @pl.when(i == 0)
        def _():
            acc[...] = jnp.zeros_like(acc)
            cnt[...] = jnp.zeros_like(cnt)

        z = jnp.dot(x_ref[0], wl_ref[:Hc, :],
                    preferred_element_type=jnp.float32)
        z = z + jnp.dot(x_ref[1], wl_ref[Hc:, :],
                        preferred_element_type=jnp.float32)
        ids = batch_ref[...]
        gids = lax.broadcasted_iota(jnp.int32, (BN_ROWS, G), 1)
        mask = (ids == gids).astype(jnp.float32)
        acc[...] += jnp.sum(mask * z, axis=0, keepdims=True)
        cnt[...] += jnp.sum(mask, axis=0, keepdims=True)

        @pl.when(i == grid - 1)
        def _():
            mean = acc[...] / jnp.maximum(cnt[...], 1.0)
            o_ref[...] = mean.reshape(G, 1) + bl_ref[0, 0]

    return pl.pallas_call(
        body,
        grid=(grid,),
        in_specs=[
            pl.BlockSpec((NC, BN_ROWS, Hc), lambda i: (0, i, 0)),
            pl.BlockSpec((H, 1), lambda i: (0, 0)),
            pl.BlockSpec((1, 1), lambda i: (0, 0)),
            pl.BlockSpec((BN_ROWS, 1), lambda i: (i, 0)),
        ],
        out_specs=pl.BlockSpec((G, 1), lambda i: (0, 0)),
        out_shape=jax.ShapeDtypeStruct((G, 1), jnp.float32),
        scratch_shapes=[
            pltpu.VMEM((1, G), jnp.float32),
            pltpu.VMEM((1, G), jnp.float32),
        ],
    )(xs, W_lin, b_lin.reshape(1, 1), batch2d)


def kernel(x, edge_index, batch, W0, b0, g0, be0, W1, b1, g1, be1,
           W2, b2, g2, be2, W3, b3, g3, be3, W_lin, b_lin):
    xp = jnp.pad(x, ((0, NP - N), (0, 0)))
    zeros = jnp.zeros((NP, D_IN), jnp.float32)
    src0 = edge_index[0].reshape(NC, NS, NCH0, CH)
    dst0 = edge_index[1].reshape(NC, NS, NCH0, CH)
    src = edge_index[0].reshape(NS, NCH, CH)
    dst = edge_index[1].reshape(NS, NCH, CH)
    batch2d = jnp.pad(batch, (0, NP - N), constant_values=G).reshape(NP, 1)

    a = _make_agg0()(xp, zeros, src0, dst0)
    h = _mlp_call(a, W0, b0, g0, be0, D_IN, sum_planes=True)

    for (W, b, g, be) in [(W1, b1, g1, be1), (W2, b2, g2, be2),
                          (W3, b3, g3, be3)]:
        a = _make_agg()(h, src, dst)
        h = _mlp_call(a, W, b, g, be, H // 2)

    return _pool_call(h, W_lin, b_lin, batch2d)


# speedup vs baseline: 9.6922x; 1.1233x over previous
import functools

import jax
import jax.numpy as jnp
from jax import lax
from jax.experimental import pallas as pl
from jax.experimental.pallas import tpu as pltpu
from jax.experimental.pallas import tpu_sc as plsc

N = 10000
NP = 10240
E = 320000
D_IN = 128
H = 256
G = 64
BN_EPS = 1e-5

NC = 2
NS = 16
EPT = E // NS
CH = 50
NCH = EPT // CH
NCH0 = E // (NC * NS) // CH
RPT = NP // NS

GC = 20
NBUF = 4
LOOK = 2


def _edge_loop(table, src_hbm_t, dst_hbm_t, sva, dva, svb, dvb, rows,
               gsems, ssems, isem, acc_sh, nch):
    npair = nch // GC // 2

    def stage(gidx, sv, dv):
        pltpu.async_copy(src_hbm_t.at[gidx], sv, isem)
        pltpu.async_copy(dst_hbm_t.at[gidx], dv, isem)

    def stage_wait(sv, dv):
        pltpu.make_async_copy(src_hbm_t.at[0], sv, isem).wait()
        pltpu.make_async_copy(dst_hbm_t.at[0], dv, isem).wait()

    def gather(sv, j, b):
        pltpu.async_copy(table.at[sv.at[j]], rows[b], gsems[b])

    def gather_wait(sv, j, b):
        pltpu.make_async_copy(table.at[sv.at[j]], rows[b], gsems[b]).wait()

    def scat(dv, j, b):
        pltpu.async_copy(rows[b], acc_sh.at[dv.at[j]], ssems[b], add=True)

    def scat_wait(dv, b):
        pltpu.make_async_copy(rows[b], acc_sh.at[dv.at[0]], ssems[b]).wait()

    def group_body(sv, dv, sv_n, dv_n, stage_idx, first_pred, next_pred):
        for j in range(GC):
            b = j % NBUF
            bg = (j + LOOK) % NBUF
            if j == 2:
                if next_pred is None:
                    stage(stage_idx, sv_n, dv_n)
                else:
                    @pl.when(next_pred)
                    def _():
                        stage(stage_idx, sv_n, dv_n)
            if j == GC - LOOK:
                if next_pred is None:
                    stage_wait(sv_n, dv_n)
                else:
                    @pl.when(next_pred)
                    def _():
                        stage_wait(sv_n, dv_n)
            if j + LOOK < GC:
                if j < NBUF - LOOK and first_pred is not None:
                    @pl.when(first_pred)
                    def _():
                        scat_wait(dv, bg)

                    gather(sv, j + LOOK, bg)
                else:
                    scat_wait(dv, bg)
                    gather(sv, j + LOOK, bg)
            else:
                if next_pred is None:
                    scat_wait(dv, bg)
                    gather(sv_n, j + LOOK - GC, bg)
                else:
                    @pl.when(next_pred)
                    def _():
                        scat_wait(dv, bg)
                        gather(sv_n, j + LOOK - GC, bg)
            gather_wait(sv, j, b)
            scat(dv, j, b)

    pltpu.sync_copy(src_hbm_t.at[0], sva)
    pltpu.sync_copy(dst_hbm_t.at[0], dva)
    gather(sva, 0, 0)
    gather(sva, 1, 1)

    def pair(m, carry):
        group_body(sva, dva, svb, dvb, 2 * m + 1,
                   first_pred=m > 0, next_pred=None)
        group_body(svb, dvb, sva, dva, 2 * m + 2,
                   first_pred=None, next_pred=m < npair - 1)
        return carry

    lax.fori_loop(0, npair, pair, 0)

    for b in range(NBUF):
        scat_wait(dvb, b)


def _sc_scratch(hc):
    return [
        pltpu.VMEM((GC, CH), jnp.int32),
        pltpu.VMEM((GC, CH), jnp.int32),
        pltpu.VMEM((GC, CH), jnp.int32),
        pltpu.VMEM((GC, CH), jnp.int32),
        pltpu.VMEM((CH, hc), jnp.float32),
        pltpu.VMEM((CH, hc), jnp.float32),
        pltpu.VMEM((CH, hc), jnp.float32),
        pltpu.VMEM((CH, hc), jnp.float32),
        pltpu.VMEM_SHARED((NP, hc), jnp.float32),
        pltpu.SemaphoreType.DMA,
        pltpu.SemaphoreType.DMA,
        pltpu.SemaphoreType.DMA,
        pltpu.SemaphoreType.DMA,
        pltpu.SemaphoreType.DMA,
        pltpu.SemaphoreType.DMA,
        pltpu.SemaphoreType.DMA,
        pltpu.SemaphoreType.DMA,
        pltpu.SemaphoreType.DMA,
    ]


@functools.lru_cache(maxsize=None)
def _make_agg0():
    mesh = plsc.VectorSubcoreMesh(core_axis_name="c", subcore_axis_name="s")

    @functools.partial(
        pl.kernel,
        mesh=mesh,
        out_type=jax.ShapeDtypeStruct((NC, NP, D_IN), jnp.float32),
        scratch_types=_sc_scratch(D_IN),
    )
    def agg0(h_hbm, z_hbm, src_hbm, dst_hbm, out_hbm, sva, dva, svb, dvb,
             rows0, rows1, rows2, rows3, acc_sh, g0, g1, g2, g3,
             s0, s1, s2, s3, isem):
        c = lax.axis_index("c")
        s = lax.axis_index("s")

        slab = pl.ds(s * RPT, RPT)

        @pl.when(c == 0)
        def _():
            pltpu.sync_copy(h_hbm.at[slab], acc_sh.at[slab])

        @pl.when(c == 1)
        def _():
            pltpu.sync_copy(z_hbm.at[slab], acc_sh.at[slab])

        plsc.subcore_barrier()
        _edge_loop(h_hbm, src_hbm.at[c, s], dst_hbm.at[c, s],
                   sva, dva, svb, dvb, (rows0, rows1, rows2, rows3),
                   (g0, g1, g2, g3), (s0, s1, s2, s3), isem, acc_sh, NCH0)
        plsc.subcore_barrier()
        pltpu.sync_copy(acc_sh.at[slab], out_hbm.at[c, slab])

    return agg0


@functools.lru_cache(maxsize=None)
def _make_agg():
    Hc = H // 2
    mesh = plsc.VectorSubcoreMesh(core_axis_name="c", subcore_axis_name="s")

    @functools.partial(
        pl.kernel,
        mesh=mesh,
        out_type=jax.ShapeDtypeStruct((NC, NP, Hc), jnp.float32),
        scratch_types=_sc_scratch(H // 2),
    )
    def agg(h_hbm, src_hbm, dst_hbm, out_hbm, sva, dva, svb, dvb,
            rows0, rows1, rows2, rows3, acc_sh, g0, g1, g2, g3,
            s0, s1, s2, s3, isem):
        c = lax.axis_index("c")
        s = lax.axis_index("s")
        table = h_hbm.at[c]

        slab = pl.ds(s * RPT, RPT)
        pltpu.sync_copy(table.at[slab], acc_sh.at[slab])
        plsc.subcore_barrier()
        _edge_loop(table, src_hbm.at[s], dst_hbm.at[s],
                   sva, dva, svb, dvb, (rows0, rows1, rows2, rows3),
                   (g0, g1, g2, g3), (s0, s1, s2, s3), isem, acc_sh, NCH)
        plsc.subcore_barrier()
        pltpu.sync_copy(acc_sh.at[slab], out_hbm.at[c, slab])

    return agg


def _mlp_call(xs, W, bvec, gvec, bevec, Hc, sum_planes=False):
    BN_ROWS = 1024
    grid = NP // BN_ROWS
    inv = 1.0 / (1.0 + BN_EPS) ** 0.5
    Ho = H // 2

    def body(x_ref, w_ref, b_ref, g_ref, be_ref, o_ref):
        x0 = x_ref[0]
        x1 = x_ref[1]
        if sum_planes:
            h = jnp.dot(x0 + x1, w_ref[...],
                        preferred_element_type=jnp.float32)
        else:
            h = jnp.dot(x0, w_ref[:Hc, :],
                        preferred_element_type=jnp.float32)
            h = h + jnp.dot(x1, w_ref[Hc:, :],
                            preferred_element_type=jnp.float32)
        h = h + b_ref[...]
        h = g_ref[...] * (h * inv) + be_ref[...]
        h = jnp.maximum(h, 0.0)
        o_ref[0] = h[:, :Ho]
        o_ref[1] = h[:, Ho:]

    return pl.pallas_call(
        body,
        grid=(grid,),
        in_specs=[
            pl.BlockSpec((NC, BN_ROWS, Hc), lambda i: (0, i, 0)),
            pl.BlockSpec((Hc if sum_planes else 2 * Hc, H),
                         lambda i: (0, 0)),
            pl.BlockSpec((1, H), lambda i: (0, 0)),
            pl.BlockSpec((1, H), lambda i: (0, 0)),
            pl.BlockSpec((1, H), lambda i: (0, 0)),
        ],
        out_specs=pl.BlockSpec((NC, BN_ROWS, Ho), lambda i: (0, i, 0)),
        out_shape=jax.ShapeDtypeStruct((NC, NP, Ho), jnp.float32),
    )(xs, W, bvec.reshape(1, H), gvec.reshape(1, H), bevec.reshape(1, H))


def _pool_call(xs, W_lin, b_lin, batch2d):
    BN_ROWS = 1024
    grid = NP // BN_ROWS
    Hc = H // 2

    def body(x_ref, wl_ref, bl_ref, batch_ref, o_ref, acc, cnt):
        i = pl.program_id(0)

        @pl.when(i == 0)
        def _():
            acc[...] = jnp.zeros_like(acc)
            cnt[...] = jnp.zeros_like(cnt)

        z = jnp.dot(x_ref[0], wl_ref[:Hc, :],
                    preferred_element_type=jnp.float32)
        z = z + jnp.dot(x_ref[1], wl_ref[Hc:, :],
                        preferred_element_type=jnp.float32)
        ids = batch_ref[...]
        gids = lax.broadcasted_iota(jnp.int32, (BN_ROWS, G), 1)
        mask = (ids == gids).astype(jnp.float32)
        acc[...] += jnp.sum(mask * z, axis=0, keepdims=True)
        cnt[...] += jnp.sum(mask, axis=0, keepdims=True)

        @pl.when(i == grid - 1)
        def _():
            mean = acc[...] / jnp.maximum(cnt[...], 1.0)
            o_ref[...] = mean.reshape(G, 1) + bl_ref[0, 0]

    return pl.pallas_call(
        body,
        grid=(grid,),
        in_specs=[
            pl.BlockSpec((NC, BN_ROWS, Hc), lambda i: (0, i, 0)),
            pl.BlockSpec((H, 1), lambda i: (0, 0)),
            pl.BlockSpec((1, 1), lambda i: (0, 0)),
            pl.BlockSpec((BN_ROWS, 1), lambda i: (i, 0)),
        ],
        out_specs=pl.BlockSpec((G, 1), lambda i: (0, 0)),
        out_shape=jax.ShapeDtypeStruct((G, 1), jnp.float32),
        scratch_shapes=[
            pltpu.VMEM((1, G), jnp.float32),
            pltpu.VMEM((1, G), jnp.float32),
        ],
    )(xs, W_lin, b_lin.reshape(1, 1), batch2d)


def kernel(x, edge_index, batch, W0, b0, g0, be0, W1, b1, g1, be1,
           W2, b2, g2, be2, W3, b3, g3, be3, W_lin, b_lin):
    xp = jnp.pad(x, ((0, NP - N), (0, 0)))
    zeros = jnp.zeros((NP, D_IN), jnp.float32)
    src0 = edge_index[0].reshape(NC, NS, NCH0 // GC, GC, CH)
    dst0 = edge_index[1].reshape(NC, NS, NCH0 // GC, GC, CH)
    src = edge_index[0].reshape(NS, NCH // GC, GC, CH)
    dst = edge_index[1].reshape(NS, NCH // GC, GC, CH)
    batch2d = jnp.pad(batch, (0, NP - N), constant_values=G).reshape(NP, 1)

    a = _make_agg0()(xp, zeros, src0, dst0)
    h = _mlp_call(a, W0, b0, g0, be0, D_IN, sum_planes=True)

    for (W, b, g, be) in [(W1, b1, g1, be1), (W2, b2, g2, be2),
                          (W3, b3, g3, be3)]:
        a = _make_agg()(h, src, dst)
        h = _mlp_call(a, W, b, g, be, H // 2)

    return _pool_call(h, W_lin, b_lin, batch2d)


# speedup vs baseline: 9.8437x; 1.0156x over previous
import functools

import jax
import jax.numpy as jnp
from jax import lax
from jax.experimental import pallas as pl
from jax.experimental.pallas import tpu as pltpu
from jax.experimental.pallas import tpu_sc as plsc

N = 10000
NP = 10240
E = 320000
D_IN = 128
H = 256
G = 64
BN_EPS = 1e-5

NC = 2
NS = 16
EPT = E // NS
CH = 50
NCH = EPT // CH
NCH0 = E // (NC * NS) // CH
RPT = NP // NS

GC = 20
NBUF = 4
LOOK = 2


def _edge_loop(table, src_hbm_t, dst_hbm_t, sva, dva, svb, dvb, rows,
               gsems, ssems, isem, acc_sh, nch):
    npair = nch // GC // 2

    def stage(gidx, sv, dv):
        pltpu.async_copy(src_hbm_t.at[gidx], sv, isem)
        pltpu.async_copy(dst_hbm_t.at[gidx], dv, isem)

    def stage_wait(sv, dv):
        pltpu.make_async_copy(src_hbm_t.at[0], sv, isem).wait()
        pltpu.make_async_copy(dst_hbm_t.at[0], dv, isem).wait()

    def gather(sv, j, b):
        pltpu.async_copy(table.at[sv.at[j]], rows[b], gsems[b])

    def gather_wait(sv, j, b):
        pltpu.make_async_copy(table.at[sv.at[j]], rows[b], gsems[b]).wait()

    def scat(dv, j, b):
        pltpu.async_copy(rows[b], acc_sh.at[dv.at[j]], ssems[b], add=True)

    def scat_wait(dv, b):
        pltpu.make_async_copy(rows[b], acc_sh.at[dv.at[0]], ssems[b]).wait()

    def group_body(sv, dv, sv_n, dv_n, stage_idx, first_pred, next_pred):
        for j in range(GC):
            b = j % NBUF
            bg = (j + LOOK) % NBUF
            if j == 2:
                if next_pred is None:
                    stage(stage_idx, sv_n, dv_n)
                else:
                    @pl.when(next_pred)
                    def _():
                        stage(stage_idx, sv_n, dv_n)
            if j == GC - LOOK:
                if next_pred is None:
                    stage_wait(sv_n, dv_n)
                else:
                    @pl.when(next_pred)
                    def _():
                        stage_wait(sv_n, dv_n)
            if j + LOOK < GC:
                if j < NBUF - LOOK and first_pred is not None:
                    @pl.when(first_pred)
                    def _():
                        scat_wait(dv, bg)

                    gather(sv, j + LOOK, bg)
                else:
                    scat_wait(dv, bg)
                    gather(sv, j + LOOK, bg)
            else:
                if next_pred is None:
                    scat_wait(dv, bg)
                    gather(sv_n, j + LOOK - GC, bg)
                else:
                    @pl.when(next_pred)
                    def _():
                        scat_wait(dv, bg)
                        gather(sv_n, j + LOOK - GC, bg)
            gather_wait(sv, j, b)
            scat(dv, j, b)

    pltpu.sync_copy(src_hbm_t.at[0], sva)
    pltpu.sync_copy(dst_hbm_t.at[0], dva)
    gather(sva, 0, 0)
    gather(sva, 1, 1)

    def pair(m, carry):
        group_body(sva, dva, svb, dvb, 2 * m + 1,
                   first_pred=m > 0, next_pred=None)
        group_body(svb, dvb, sva, dva, 2 * m + 2,
                   first_pred=None, next_pred=m < npair - 1)
        return carry

    lax.fori_loop(0, npair, pair, 0)

    for b in range(NBUF):
        scat_wait(dvb, b)


def _sc_scratch(hc):
    return [
        pltpu.VMEM((GC, CH), jnp.int32),
        pltpu.VMEM((GC, CH), jnp.int32),
        pltpu.VMEM((GC, CH), jnp.int32),
        pltpu.VMEM((GC, CH), jnp.int32),
        pltpu.VMEM((CH, hc), jnp.float32),
        pltpu.VMEM((CH, hc), jnp.float32),
        pltpu.VMEM((CH, hc), jnp.float32),
        pltpu.VMEM((CH, hc), jnp.float32),
        pltpu.VMEM_SHARED((NP, hc), jnp.float32),
        pltpu.SemaphoreType.DMA,
        pltpu.SemaphoreType.DMA,
        pltpu.SemaphoreType.DMA,
        pltpu.SemaphoreType.DMA,
        pltpu.SemaphoreType.DMA,
        pltpu.SemaphoreType.DMA,
        pltpu.SemaphoreType.DMA,
        pltpu.SemaphoreType.DMA,
        pltpu.SemaphoreType.DMA,
    ]


@functools.lru_cache(maxsize=None)
def _make_agg0():
    mesh = plsc.VectorSubcoreMesh(core_axis_name="c", subcore_axis_name="s")

    @functools.partial(
        pl.kernel,
        mesh=mesh,
        out_type=jax.ShapeDtypeStruct((NC, NP, D_IN), jnp.float32),
        scratch_types=_sc_scratch(D_IN),
    )
    def agg0(h_hbm, z_hbm, src_hbm, dst_hbm, out_hbm, sva, dva, svb, dvb,
             rows0, rows1, rows2, rows3, acc_sh, g0, g1, g2, g3,
             s0, s1, s2, s3, isem):
        c = lax.axis_index("c")
        s = lax.axis_index("s")

        slab = pl.ds(s * RPT, RPT)

        @pl.when(c == 0)
        def _():
            pltpu.sync_copy(h_hbm.at[slab], acc_sh.at[slab])

        @pl.when(c == 1)
        def _():
            pltpu.sync_copy(z_hbm.at[slab], acc_sh.at[slab])

        plsc.subcore_barrier()
        _edge_loop(h_hbm, src_hbm.at[c, s], dst_hbm.at[c, s],
                   sva, dva, svb, dvb, (rows0, rows1, rows2, rows3),
                   (g0, g1, g2, g3), (s0, s1, s2, s3), isem, acc_sh, NCH0)
        plsc.subcore_barrier()
        pltpu.sync_copy(acc_sh.at[slab], out_hbm.at[c, slab])

    return agg0


@functools.lru_cache(maxsize=None)
def _make_agg():
    Hc = H // 2
    mesh = plsc.VectorSubcoreMesh(core_axis_name="c", subcore_axis_name="s")

    @functools.partial(
        pl.kernel,
        mesh=mesh,
        out_type=jax.ShapeDtypeStruct((NC, NP, Hc), jnp.float32),
        scratch_types=_sc_scratch(H // 2),
    )
    def agg(h_hbm, src_hbm, dst_hbm, out_hbm, sva, dva, svb, dvb,
            rows0, rows1, rows2, rows3, acc_sh, g0, g1, g2, g3,
            s0, s1, s2, s3, isem):
        c = lax.axis_index("c")
        s = lax.axis_index("s")
        table = h_hbm.at[c]

        slab = pl.ds(s * RPT, RPT)
        pltpu.sync_copy(table.at[slab], acc_sh.at[slab])
        plsc.subcore_barrier()
        _edge_loop(table, src_hbm.at[s], dst_hbm.at[s],
                   sva, dva, svb, dvb, (rows0, rows1, rows2, rows3),
                   (g0, g1, g2, g3), (s0, s1, s2, s3), isem, acc_sh, NCH)
        plsc.subcore_barrier()
        pltpu.sync_copy(acc_sh.at[slab], out_hbm.at[c, slab])

    return agg


def _mlp_call(xs, W, bvec, gvec, bevec, Hc, sum_planes=False):
    BN_ROWS = 1024
    grid = NP // BN_ROWS
    inv = 1.0 / (1.0 + BN_EPS) ** 0.5
    Ho = H // 2

    def body(x_ref, w_ref, b_ref, g_ref, be_ref, o_ref):
        x0 = x_ref[0]
        x1 = x_ref[1]
        if sum_planes:
            h = jnp.dot(x0 + x1, w_ref[...],
                        preferred_element_type=jnp.float32)
        else:
            h = jnp.dot(x0, w_ref[:Hc, :],
                        preferred_element_type=jnp.float32)
            h = h + jnp.dot(x1, w_ref[Hc:, :],
                            preferred_element_type=jnp.float32)
        h = h + b_ref[...]
        h = g_ref[...] * (h * inv) + be_ref[...]
        h = jnp.maximum(h, 0.0)
        o_ref[0] = h[:, :Ho]
        o_ref[1] = h[:, Ho:]

    return pl.pallas_call(
        body,
        grid=(grid,),
        in_specs=[
            pl.BlockSpec((NC, BN_ROWS, Hc), lambda i: (0, i, 0)),
            pl.BlockSpec((Hc if sum_planes else 2 * Hc, H),
                         lambda i: (0, 0)),
            pl.BlockSpec((1, H), lambda i: (0, 0)),
            pl.BlockSpec((1, H), lambda i: (0, 0)),
            pl.BlockSpec((1, H), lambda i: (0, 0)),
        ],
        out_specs=pl.BlockSpec((NC, BN_ROWS, Ho), lambda i: (0, i, 0)),
        out_shape=jax.ShapeDtypeStruct((NC, NP, Ho), jnp.float32),
    )(xs, W, bvec.reshape(1, H), gvec.reshape(1, H), bevec.reshape(1, H))


def _mlp_pool_call(xs, W, bvec, gvec, bevec, W_lin, b_lin, batch2d):
    BN_ROWS = 1024
    grid = NP // BN_ROWS
    inv = 1.0 / (1.0 + BN_EPS) ** 0.5
    Hc = H // 2

    def body(x_ref, w_ref, b_ref, g_ref, be_ref, wl_ref, bl_ref, batch_ref,
             o_ref, acc, cnt):
        i = pl.program_id(0)

        @pl.when(i == 0)
        def _():
            acc[...] = jnp.zeros_like(acc)
            cnt[...] = jnp.zeros_like(cnt)

        h = jnp.dot(x_ref[0], w_ref[:Hc, :],
                    preferred_element_type=jnp.float32)
        h = h + jnp.dot(x_ref[1], w_ref[Hc:, :],
                        preferred_element_type=jnp.float32)
        h = h + b_ref[...]
        h = g_ref[...] * (h * inv) + be_ref[...]
        h = jnp.maximum(h, 0.0)
        z = jnp.dot(h, wl_ref[...],
                    preferred_element_type=jnp.float32)
        ids = batch_ref[...]
        gids = lax.broadcasted_iota(jnp.int32, (BN_ROWS, G), 1)
        mask = (ids == gids).astype(jnp.float32)
        acc[...] += jnp.sum(mask * z, axis=0, keepdims=True)
        cnt[...] += jnp.sum(mask, axis=0, keepdims=True)

        @pl.when(i == grid - 1)
        def _():
            mean = acc[...] / jnp.maximum(cnt[...], 1.0)
            o_ref[...] = mean.reshape(G, 1) + bl_ref[0, 0]

    return pl.pallas_call(
        body,
        grid=(grid,),
        in_specs=[
            pl.BlockSpec((NC, BN_ROWS, Hc), lambda i: (0, i, 0)),
            pl.BlockSpec((H, H), lambda i: (0, 0)),
            pl.BlockSpec((1, H), lambda i: (0, 0)),
            pl.BlockSpec((1, H), lambda i: (0, 0)),
            pl.BlockSpec((1, H), lambda i: (0, 0)),
            pl.BlockSpec((H, 1), lambda i: (0, 0)),
            pl.BlockSpec((1, 1), lambda i: (0, 0)),
            pl.BlockSpec((BN_ROWS, 1), lambda i: (i, 0)),
        ],
        out_specs=pl.BlockSpec((G, 1), lambda i: (0, 0)),
        out_shape=jax.ShapeDtypeStruct((G, 1), jnp.float32),
        scratch_shapes=[
            pltpu.VMEM((1, G), jnp.float32),
            pltpu.VMEM((1, G), jnp.float32),
        ],
    )(xs, W, bvec.reshape(1, H), gvec.reshape(1, H), bevec.reshape(1, H),
      W_lin, b_lin.reshape(1, 1), batch2d)


def kernel(x, edge_index, batch, W0, b0, g0, be0, W1, b1, g1, be1,
           W2, b2, g2, be2, W3, b3, g3, be3, W_lin, b_lin):
    xp = jnp.pad(x, ((0, NP - N), (0, 0)))
    zeros = jnp.zeros((NP, D_IN), jnp.float32)
    src0 = edge_index[0].reshape(NC, NS, NCH0 // GC, GC, CH)
    dst0 = edge_index[1].reshape(NC, NS, NCH0 // GC, GC, CH)
    src = edge_index[0].reshape(NS, NCH // GC, GC, CH)
    dst = edge_index[1].reshape(NS, NCH // GC, GC, CH)
    batch2d = jnp.pad(batch, (0, NP - N), constant_values=G).reshape(NP, 1)

    a = _make_agg0()(xp, zeros, src0, dst0)
    h = _mlp_call(a, W0, b0, g0, be0, D_IN, sum_planes=True)

    for (W, b, g, be) in [(W1, b1, g1, be1), (W2, b2, g2, be2)]:
        a = _make_agg()(h, src, dst)
        h = _mlp_call(a, W, b, g, be, H // 2)

    a = _make_agg()(h, src, dst)
    return _mlp_pool_call(a, W3, b3, g3, be3, W_lin, b_lin, batch2d)
